# baseline (speedup 1.0000x reference)
import functools

import jax
import jax.numpy as jnp
from jax import lax
from jax.experimental import pallas as pl
from jax.experimental.pallas import tpu as pltpu
from jax.experimental.pallas import tpu_sc as plsc

VOCAB = 1000000
EMB_DIM = 32
BATCH = 4096
SEQ_LEN = 50

NC, NS = 2, 16
NW = NC * NS
BTILE = BATCH // NW
NGRP = EMB_DIM // 8
NH = BTILE // 16

CH = 512
FULL_CHUNKS = (VOCAB // CH // NW) * NW
PER_W = FULL_CHUNKS // NW
TAIL1_V0 = FULL_CHUNKS * CH
TAIL2_V0 = TAIL1_V0 + CH


def _make_reformat():
    mesh = plsc.VectorSubcoreMesh(core_axis_name="c", subcore_axis_name="s")

    @functools.partial(
        pl.kernel,
        mesh=mesh,
        out_type=jax.ShapeDtypeStruct((VOCAB // 4, 128), jnp.float32),
        scratch_types=[
            pltpu.VMEM((2, EMB_DIM, CH), jnp.float32),
            pltpu.VMEM((2, CH // 4, 128), jnp.float32),
            pltpu.VMEM((EMB_DIM, 64), jnp.float32),
            [pltpu.SemaphoreType.DMA] * 2,
            [pltpu.SemaphoreType.DMA] * 2,
        ],
        compiler_params=pltpu.CompilerParams(use_tc_tiling_on_sc=True,
                                             needs_layout_passes=False),
    )
    def reformat(tt_hbm, tail_hbm, out_hbm, in_v, o_v, tail_v, isem, osem):
        w = lax.axis_index("s") * NC + lax.axis_index("c")
        lane = lax.iota(jnp.int32, 16)
        cvecs = [lane, lane + 16]

        def in_cp(v0, buf, n=CH):
            return pltpu.make_async_copy(
                tt_hbm.at[:, pl.ds(pl.multiple_of(v0, 128), n)],
                in_v.at[buf, :, pl.ds(0, n)], isem[buf])

        def out_cp(v0, buf, n=CH):
            return pltpu.make_async_copy(
                o_v.at[buf, pl.ds(0, n // 4), :],
                out_hbm.at[pl.ds(pl.multiple_of(v0 // 4, 8), n // 4)],
                osem[buf])

        def transform(buf, nj=CH // 4, src=None):
            src_ref = in_v.at[buf] if src is None else src

            def jstep(j8, _):
                for jj in range(8):
                    j = j8 * 8 + jj
                    for q in range(4):
                        vsp = jnp.full((16,), 0, jnp.int32) + (4 * j + q)
                        for k in range(2):
                            o_v[buf, j, pl.ds(32 * q + 16 * k, 16)] = (
                                plsc.load_gather(src_ref, [cvecs[k], vsp]))
                return _
            lax.fori_loop(0, nj // 8, jstep, 0)

        def v0_of(k):
            return (w * PER_W + k) * CH

        in_cp(v0_of(0), 0).start()

        def body(t, carry):
            k0 = 2 * t
            in_cp(v0_of(k0 + 1), 1).start()
            in_cp(v0_of(k0), 0).wait()

            @pl.when(t >= 1)
            def _():
                out_cp(v0_of(k0), 0).wait()
            transform(0)
            out_cp(v0_of(k0), 0).start()

            @pl.when(k0 + 2 < PER_W)
            def _():
                in_cp(v0_of(k0 + 2), 0).start()
            in_cp(v0_of(k0 + 1), 1).wait()

            @pl.when(t >= 1)
            def _():
                out_cp(v0_of(k0 + 1), 1).wait()
            transform(1)
            out_cp(v0_of(k0 + 1), 1).start()
            return carry

        lax.fori_loop(0, PER_W // 2, body, 0)
        klast = PER_W - 1
        in_cp(v0_of(klast), 0).wait()
        out_cp(v0_of(klast), 0).wait()
        transform(0)
        out_cp(v0_of(klast), 0).start()

        @pl.when(w == NW - 1)
        def _():
            out_cp(v0_of(klast - 1), 1).wait()
            in_cp(TAIL1_V0, 1).start()
            in_cp(TAIL1_V0, 1).wait()
            transform(1)
            out_cp(TAIL1_V0, 1).start()
            out_cp(v0_of(klast), 0).wait()
            tail_cp = pltpu.make_async_copy(tail_hbm, tail_v, isem[0])
            tail_cp.start()
            tail_cp.wait()
            transform(0, 16, src=tail_v)
            out_cp(TAIL1_V0, 1).wait()
            out_cp(TAIL2_V0, 0, 64).start()
            out_cp(TAIL2_V0, 0, 64).wait()

        @pl.when(w != NW - 1)
        def _():
            out_cp(v0_of(klast - 1), 1).wait()
            out_cp(v0_of(klast), 0).wait()

    return reformat


def _make_gather():
    mesh = plsc.VectorSubcoreMesh(core_axis_name="c", subcore_axis_name="s")

    @functools.partial(
        pl.kernel,
        mesh=mesh,
        out_type=jax.ShapeDtypeStruct((SEQ_LEN, NGRP, NW, 8, BTILE),
                                      jnp.float32),
        scratch_types=[
            pltpu.VMEM((SEQ_LEN, BTILE), jnp.int32),
            pltpu.VMEM((2, BTILE, EMB_DIM), jnp.float32),
            pltpu.VMEM((2, NGRP, 8, BTILE), jnp.float32),
            [pltpu.SemaphoreType.DMA] * 2,
            [pltpu.SemaphoreType.DMA] * 2,
        ],
        compiler_params=pltpu.CompilerParams(use_tc_tiling_on_sc=False,
                                             needs_layout_passes=False),
    )
    def gather_kernel(idx_hbm, table_hbm, out_hbm, idx_v, rows_v, out_v,
                      gsem, osem):
        w = lax.axis_index("s") * NC + lax.axis_index("c")
        b0 = w * BTILE
        pltpu.sync_copy(idx_hbm.at[:, pl.ds(b0, BTILE)], idx_v)

        lane = lax.iota(jnp.int32, 16)
        row_ids = [lane + (16 * h) for h in range(NH)]

        def gather_s(s, buf):
            return pltpu.make_async_copy(
                table_hbm.at[idx_v.at[s]], rows_v.at[buf], gsem[buf])

        def out_s(s, buf):
            return pltpu.make_async_copy(
                out_v.at[buf], out_hbm.at[s, :, w], osem[buf])

        def transpose(buf):
            for g in range(NGRP):
                for ci2 in range(0, 8, 2):
                    vals = []
                    for ci in (ci2, ci2 + 1):
                        col = jnp.full((16,), g * 8 + ci, jnp.int32)
                        for h in range(NH):
                            vals.append(plsc.load_gather(
                                rows_v.at[buf], [row_ids[h], col]))
                    for k, ci in enumerate((ci2, ci2 + 1)):
                        for h in range(NH):
                            out_v[buf, g, ci, pl.ds(16 * h, 16)] = (
                                vals[k * 8 + h])

        gather_s(0, 0).start()

        def body(t, carry):
            s0 = 2 * t
            gather_s(s0 + 1, 1).start()
            gather_s(s0, 0).wait()

            @pl.when(t >= 1)
            def _():
                out_s(s0, 0).wait()
            transpose(0)
            out_s(s0, 0).start()

            s1 = s0 + 1

            @pl.when(s1 + 1 < SEQ_LEN)
            def _():
                gather_s(s1 + 1, 0).start()

            @pl.when(t >= 1)
            def _():
                out_s(s1, 1).wait()
            gather_s(s1, 1).wait()
            transpose(1)
            out_s(s1, 1).start()
            return carry

        lax.fori_loop(0, SEQ_LEN // 2, body, 0)
        out_s(SEQ_LEN - 2, 0).wait()
        out_s(SEQ_LEN - 1, 1).wait()

    return gather_kernel


_reformat = _make_reformat()
_gather = _make_gather()


@jax.jit
def kernel(indices, table):
    idx_t = jnp.swapaxes(indices, 0, 1).astype(jnp.int32)
    tt = jnp.swapaxes(table, 0, 1)
    tail = lax.slice(tt, (0, TAIL2_V0), (EMB_DIM, VOCAB))
    tview = _reformat(tt, tail)
    tlin = tview.reshape(VOCAB, EMB_DIM)
    out5 = _gather(idx_t, tlin)
    return out5.transpose(2, 4, 0, 1, 3).reshape(BATCH, SEQ_LEN, EMB_DIM)


# speedup vs baseline: 1.4794x; 1.4794x over previous
import functools

import jax
import jax.numpy as jnp
from jax import lax
from jax.experimental import pallas as pl
from jax.experimental.pallas import tpu as pltpu
from jax.experimental.pallas import tpu_sc as plsc

VOCAB = 1000000
EMB_DIM = 32
BATCH = 4096
SEQ_LEN = 50

NC, NS = 2, 16
NW = NC * NS
BTILE = BATCH // NW
NGRP = EMB_DIM // 8
NH = BTILE // 16

CH = 512
FULL_CHUNKS = (VOCAB // CH // NW) * NW
PER_W = FULL_CHUNKS // NW
TAIL1_V0 = FULL_CHUNKS * CH
TAIL2_V0 = TAIL1_V0 + CH


def _make_reformat():
    mesh = plsc.VectorSubcoreMesh(core_axis_name="c", subcore_axis_name="s")

    @functools.partial(
        pl.kernel,
        mesh=mesh,
        out_type=jax.ShapeDtypeStruct((VOCAB * EMB_DIM,), jnp.float32),
        scratch_types=[
            pltpu.VMEM((2, EMB_DIM, CH), jnp.float32),
            pltpu.VMEM((CH * EMB_DIM,), jnp.float32),
            pltpu.VMEM((CH * EMB_DIM,), jnp.float32),
            pltpu.VMEM((EMB_DIM, 64), jnp.float32),
            [pltpu.SemaphoreType.DMA] * 2,
            [pltpu.SemaphoreType.DMA] * 2,
        ],
        compiler_params=pltpu.CompilerParams(use_tc_tiling_on_sc=True,
                                             needs_layout_passes=False),
    )
    def reformat(tt_hbm, tail_hbm, out_hbm, in_v, o_v0, o_v1, tail_v,
                 isem, osem):
        o_bufs = [o_v0, o_v1]
        w = lax.axis_index("s") * NC + lax.axis_index("c")
        lane = lax.iota(jnp.int32, 16)
        splat = ((lane >> 2) << 7) + ((lane & 3) << 5)

        def in_cp(v0, buf, n=CH):
            return pltpu.make_async_copy(
                tt_hbm.at[:, pl.ds(pl.multiple_of(v0, 128), n)],
                in_v.at[buf, :, pl.ds(0, n)], isem[buf])

        def out_cp(v0, buf, n=CH):
            return pltpu.make_async_copy(
                o_bufs[buf].at[pl.ds(0, n * EMB_DIM)],
                out_hbm.at[pl.ds(pl.multiple_of(v0 * EMB_DIM, 1024),
                                 n * EMB_DIM)],
                osem[buf])

        def transform(buf, nv=CH, src=None):
            src_ref = in_v.at[buf] if src is None else src

            def mstep(m, _):
                base = splat + (m << 9)
                for c2 in range(0, EMB_DIM, 8):
                    vals = []
                    for c in range(c2, c2 + 8):
                        vals.append(src_ref[c, pl.ds(16 * m, 16)])
                    for k, c in enumerate(range(c2, c2 + 8)):
                        plsc.store_scatter(o_bufs[buf], [base + c], vals[k])
                return _
            lax.fori_loop(0, nv // 16, mstep, 0)

        def v0_of(k):
            return (w * PER_W + k) * CH

        in_cp(v0_of(0), 0).start()

        def body(t, carry):
            k0 = 2 * t
            in_cp(v0_of(k0 + 1), 1).start()
            in_cp(v0_of(k0), 0).wait()

            @pl.when(t >= 1)
            def _():
                out_cp(v0_of(k0), 0).wait()
            transform(0)
            out_cp(v0_of(k0), 0).start()

            @pl.when(k0 + 2 < PER_W)
            def _():
                in_cp(v0_of(k0 + 2), 0).start()
            in_cp(v0_of(k0 + 1), 1).wait()

            @pl.when(t >= 1)
            def _():
                out_cp(v0_of(k0 + 1), 1).wait()
            transform(1)
            out_cp(v0_of(k0 + 1), 1).start()
            return carry

        lax.fori_loop(0, PER_W // 2, body, 0)
        klast = PER_W - 1
        in_cp(v0_of(klast), 0).wait()
        out_cp(v0_of(klast), 0).wait()
        transform(0)
        out_cp(v0_of(klast), 0).start()

        @pl.when(w == NW - 1)
        def _():
            out_cp(v0_of(klast - 1), 1).wait()
            in_cp(TAIL1_V0, 1).start()
            in_cp(TAIL1_V0, 1).wait()
            transform(1)
            out_cp(TAIL1_V0, 1).start()
            out_cp(v0_of(klast), 0).wait()
            tail_cp = pltpu.make_async_copy(tail_hbm, tail_v, isem[0])
            tail_cp.start()
            tail_cp.wait()
            transform(0, 64, src=tail_v)
            out_cp(TAIL1_V0, 1).wait()
            out_cp(TAIL2_V0, 0, 64).start()
            out_cp(TAIL2_V0, 0, 64).wait()

        @pl.when(w != NW - 1)
        def _():
            out_cp(v0_of(klast - 1), 1).wait()
            out_cp(v0_of(klast), 0).wait()

    return reformat


def _make_gather():
    mesh = plsc.VectorSubcoreMesh(core_axis_name="c", subcore_axis_name="s")

    @functools.partial(
        pl.kernel,
        mesh=mesh,
        out_type=jax.ShapeDtypeStruct((SEQ_LEN, NGRP, NW, 8, BTILE),
                                      jnp.float32),
        scratch_types=[
            pltpu.VMEM((SEQ_LEN, BTILE), jnp.int32),
            pltpu.VMEM((2, BTILE, EMB_DIM), jnp.float32),
            pltpu.VMEM((2, NGRP, 8, BTILE), jnp.float32),
            [pltpu.SemaphoreType.DMA] * 2,
            [pltpu.SemaphoreType.DMA] * 2,
        ],
        compiler_params=pltpu.CompilerParams(use_tc_tiling_on_sc=False,
                                             needs_layout_passes=False),
    )
    def gather_kernel(idx_hbm, table_hbm, out_hbm, idx_v, rows_v, out_v,
                      gsem, osem):
        w = lax.axis_index("s") * NC + lax.axis_index("c")
        b0 = w * BTILE
        pltpu.sync_copy(idx_hbm.at[:, pl.ds(b0, BTILE)], idx_v)

        lane = lax.iota(jnp.int32, 16)
        row_ids = [lane + (16 * h) for h in range(NH)]

        def gather_s(s, buf):
            return pltpu.make_async_copy(
                table_hbm.at[idx_v.at[s]], rows_v.at[buf], gsem[buf])

        def out_s(s, buf):
            return pltpu.make_async_copy(
                out_v.at[buf], out_hbm.at[s, :, w], osem[buf])

        def transpose(buf):
            for g in range(NGRP):
                for ci2 in range(0, 8, 2):
                    vals = []
                    for ci in (ci2, ci2 + 1):
                        col = jnp.full((16,), g * 8 + ci, jnp.int32)
                        for h in range(NH):
                            vals.append(plsc.load_gather(
                                rows_v.at[buf], [row_ids[h], col]))
                    for k, ci in enumerate((ci2, ci2 + 1)):
                        for h in range(NH):
                            out_v[buf, g, ci, pl.ds(16 * h, 16)] = (
                                vals[k * 8 + h])

        gather_s(0, 0).start()

        def body(t, carry):
            s0 = 2 * t
            gather_s(s0 + 1, 1).start()
            gather_s(s0, 0).wait()

            @pl.when(t >= 1)
            def _():
                out_s(s0, 0).wait()
            transpose(0)
            out_s(s0, 0).start()

            s1 = s0 + 1

            @pl.when(s1 + 1 < SEQ_LEN)
            def _():
                gather_s(s1 + 1, 0).start()

            @pl.when(t >= 1)
            def _():
                out_s(s1, 1).wait()
            gather_s(s1, 1).wait()
            transpose(1)
            out_s(s1, 1).start()
            return carry

        lax.fori_loop(0, SEQ_LEN // 2, body, 0)
        out_s(SEQ_LEN - 2, 0).wait()
        out_s(SEQ_LEN - 1, 1).wait()

    return gather_kernel


_reformat = _make_reformat()
_gather = _make_gather()


@jax.jit
def kernel(indices, table):
    idx_t = jnp.swapaxes(indices, 0, 1).astype(jnp.int32)
    tt = jnp.swapaxes(table, 0, 1)
    tail = lax.slice(tt, (0, TAIL2_V0), (EMB_DIM, VOCAB))
    tview = _reformat(tt, tail)
    tlin = tview.reshape(VOCAB, EMB_DIM)
    out5 = _gather(idx_t, tlin)
    return out5.transpose(2, 4, 0, 1, 3).reshape(BATCH, SEQ_LEN, EMB_DIM)
